# Initial kernel scaffold; baseline (speedup 1.0000x reference)
import jax
import jax.numpy as jnp
from jax.experimental import pallas as pl


def kernel(u, edge_index, neighbor_all, emb_id, att_fc1_w, att_fc1_b, att_fc2_w, att_fc2_b, w, fc1_w, fc1_b, fc2_w, fc2_b):
    raise NotImplementedError("write your pallas kernel here")


# speedup vs baseline: 1.7047x; 1.7047x over previous
import functools

import jax
import jax.numpy as jnp
from jax import lax
from jax.experimental import pallas as pl
from jax.experimental.pallas import tpu as pltpu
from jax.experimental.pallas import tpu_sc as plsc

N = 10000
H = 128
E = 320000
D = 32
OUT = 128

NW = 32
C = 128
EPT = 10112
NCHUNK = EPT // C
E_PAD = NW * EPT
N_PAD = E_PAD // D
NPT = N_PAD // NW
EXT = 2504 * 128

_mesh = plsc.VectorSubcoreMesh(core_axis_name="c", subcore_axis_name="s")


def _wid():
    return lax.axis_index("s") * 2 + lax.axis_index("c")


def _edge_h_body(t_hbm, k_hbm, i_hbm, h_hbm, idxk_v, idxi_v, tk_v, ti_v, h_v, sem):
    wid = _wid()

    def chunk(c, carry):
        ebase = wid * EPT + c * C
        pltpu.sync_copy(k_hbm.at[pl.ds(ebase, C)], idxk_v)
        pltpu.sync_copy(i_hbm.at[pl.ds(ebase, C)], idxi_v)
        d1 = pltpu.async_copy(t_hbm.at[idxk_v], tk_v, sem)
        d2 = pltpu.async_copy(t_hbm.at[idxi_v], ti_v, sem)
        d1.wait()
        d2.wait()

        def row(r, rc):
            for j in range(8):
                du = tk_v[r, pl.ds(j * 16, 16)] - ti_v[r, pl.ds(j * 16, 16)]
                dx = tk_v[r, pl.ds(128 + j * 16, 16)] - ti_v[r, pl.ds(128 + j * 16, 16)]
                h_v[r, pl.ds(j * 16, 16)] = du * dx
            return rc

        lax.fori_loop(0, C, row, 0)
        pltpu.sync_copy(h_v, h_hbm.at[pl.ds(ebase, C)])
        return carry

    lax.fori_loop(0, NCHUNK, chunk, 0)


@functools.partial(jax.jit, static_argnames=())
def _edge_h(t, kp, ip):
    return pl.kernel(
        _edge_h_body,
        out_type=jax.ShapeDtypeStruct((E_PAD, H), jnp.float32),
        mesh=_mesh,
        scratch_types=[
            pltpu.VMEM((C,), jnp.int32),
            pltpu.VMEM((C,), jnp.int32),
            pltpu.VMEM((C, 2 * H), jnp.float32),
            pltpu.VMEM((C, 2 * H), jnp.float32),
            pltpu.VMEM((C, H), jnp.float32),
            pltpu.SemaphoreType.DMA,
        ],
    )(t, kp, ip)


def _mlp_body(h_ref, w1t_ref, b1_ref, w2_ref, out_ref):
    h1 = jnp.dot(h_ref[...], w1t_ref[...], preferred_element_type=jnp.float32)
    h1 = jnp.maximum(h1 + b1_ref[...], 0.0)
    lg = jnp.dot(h1, w2_ref[...], preferred_element_type=jnp.float32)
    out_ref[...] = lg.reshape(out_ref.shape)


def _edge_mlp(h, w1t, b1, w2col):
    eblk = 2048
    grid = (E_PAD // eblk,)
    return pl.pallas_call(
        _mlp_body,
        grid=grid,
        in_specs=[
            pl.BlockSpec((eblk, H), lambda i: (i, 0)),
            pl.BlockSpec((H, H), lambda i: (0, 0)),
            pl.BlockSpec((1, H), lambda i: (0, 0)),
            pl.BlockSpec((H, 1), lambda i: (0, 0)),
        ],
        out_specs=pl.BlockSpec((eblk // 128, 128), lambda i: (i, 0)),
        out_shape=jax.ShapeDtypeStruct((E_PAD // 128, 128), jnp.float32),
    )(h, w1t, b1, w2col)


def _softmax_body(l_ref, a_ref):
    l = l_ref[...]
    m = jnp.max(l)
    e = jnp.exp(l - m)
    a_ref[...] = e / jnp.sum(e)


def _softmax(logits_valid):
    return pl.pallas_call(
        _softmax_body,
        out_shape=jax.ShapeDtypeStruct((E // 128, 128), jnp.float32),
    )(logits_valid)


def _agg_body(nb_hbm, al_hbm, ke_hbm, x_hbm, agg_hbm,
              idxe_v, aval_v, kidx_v, rows_v, acc_v, sem):
    wid = _wid()

    def chunk(c, carry):
        sbase = wid * EPT + c * C
        pltpu.sync_copy(nb_hbm.at[pl.ds(sbase, C)], idxe_v)
        d1 = pltpu.async_copy(al_hbm.at[idxe_v], aval_v, sem)
        d2 = pltpu.async_copy(ke_hbm.at[idxe_v], kidx_v, sem)
        d1.wait()
        d2.wait()
        pltpu.async_copy(x_hbm.at[kidx_v], rows_v, sem).wait()
        for nn in range(4):
            acc = [jnp.zeros((16,), jnp.float32) for _ in range(8)]
            for g in range(2):
                avec = aval_v[pl.ds((nn * 2 + g) * 16, 16)]
                for t in range(16):
                    s = nn * D + g * 16 + t
                    a = avec[t]
                    for j in range(8):
                        acc[j] = acc[j] + a * rows_v[s, pl.ds(j * 16, 16)]
            for j in range(8):
                acc_v[nn, pl.ds(j * 16, 16)] = acc[j]
        node_base = wid * NPT + c * 4
        pltpu.sync_copy(acc_v, agg_hbm.at[pl.ds(node_base, 4)])
        return carry

    lax.fori_loop(0, NCHUNK, chunk, 0)


def _aggregate(neigh_flat, alpha_ext, kext, x):
    return pl.kernel(
        _agg_body,
        out_type=jax.ShapeDtypeStruct((N_PAD, H), jnp.float32),
        mesh=_mesh,
        scratch_types=[
            pltpu.VMEM((C,), jnp.int32),
            pltpu.VMEM((C,), jnp.float32),
            pltpu.VMEM((C,), jnp.int32),
            pltpu.VMEM((C, H), jnp.float32),
            pltpu.VMEM((4, H), jnp.float32),
            pltpu.SemaphoreType.DMA,
        ],
    )(neigh_flat, alpha_ext, kext, x)


def _head_body(x_ref, agg_ref, w0_ref, w1_ref, fc1t_ref, fc1b_ref,
               fc2t_ref, fc2b_ref, out_ref):
    x2 = jnp.dot(x_ref[...], w0_ref[...], preferred_element_type=jnp.float32)
    x2 = x2 + jnp.dot(agg_ref[...], w1_ref[...], preferred_element_type=jnp.float32)
    h2 = jnp.dot(x2, fc1t_ref[...], preferred_element_type=jnp.float32)
    h2 = jnp.maximum(h2 + fc1b_ref[...], 0.0)
    o = jnp.dot(h2, fc2t_ref[...], preferred_element_type=jnp.float32)
    out_ref[...] = o + fc2b_ref[...]


def _head(x, agg, w0, w1, fc1t, fc1b, fc2t, fc2b):
    nblk = 1000
    grid = (N // nblk,)
    wspec = pl.BlockSpec((H, H), lambda i: (0, 0))
    bspec = pl.BlockSpec((1, H), lambda i: (0, 0))
    return pl.pallas_call(
        _head_body,
        grid=grid,
        in_specs=[
            pl.BlockSpec((nblk, H), lambda i: (i, 0)),
            pl.BlockSpec((nblk, H), lambda i: (i, 0)),
            wspec, wspec, wspec, bspec,
            pl.BlockSpec((H, OUT), lambda i: (0, 0)),
            pl.BlockSpec((1, OUT), lambda i: (0, 0)),
        ],
        out_specs=pl.BlockSpec((nblk, OUT), lambda i: (i, 0)),
        out_shape=jax.ShapeDtypeStruct((N, OUT), jnp.float32),
    )(x, agg, w0, w1, fc1t, fc1b, fc2t, fc2b)


def kernel(u, edge_index, neighbor_all, emb_id,
           att_fc1_w, att_fc1_b, att_fc2_w, att_fc2_b,
           w, fc1_w, fc1_b, fc2_w, fc2_b):
    x = emb_id
    k = edge_index[0]
    i = edge_index[1]

    t = jnp.concatenate([u, x], axis=1)
    kp = jnp.pad(k, (0, E_PAD - E))
    ip = jnp.pad(i, (0, E_PAD - E))
    h = _edge_h(t, kp, ip)

    logits = _edge_mlp(h, att_fc1_w.T, att_fc1_b.reshape(1, H),
                       att_fc2_w.T)
    alpha = _softmax(logits[: E // 128])

    alpha_ext = jnp.pad(alpha.reshape(-1), (0, EXT - E))
    kext = jnp.pad(k, (0, EXT - E))
    neigh = jnp.pad(neighbor_all, ((0, N_PAD - N), (0, 0)),
                    constant_values=E).reshape(-1)
    agg = _aggregate(neigh, alpha_ext, kext, x)

    return _head(x, agg[:N], w[0], w[1], fc1_w.T,
                 fc1_b.reshape(1, H), fc2_w.T, fc2_b.reshape(1, OUT))
